# baseline (speedup 1.0000x reference)
import functools

import jax
import jax.numpy as jnp
from jax import lax
from jax.experimental import pallas as pl
from jax.experimental.pallas import tpu as pltpu
from jax.experimental.pallas import tpu_sc as plsc

NUM_ROWS = 16384
NUM_COLS = 26
DIM = 32

B_TOTAL = NUM_ROWS * NUM_COLS
NUM_EMB = 1000000
NC = 2
NS = 16
NW = NC * NS
B_PER_W = B_TOTAL // NW
CHUNK = 128
N_CHUNKS = B_PER_W // CHUNK
G2 = N_CHUNKS // 2
LANES = 16

_mesh = plsc.VectorSubcoreMesh(core_axis_name="c", subcore_axis_name="s")


@functools.partial(
    pl.kernel,
    out_type=jax.ShapeDtypeStruct((NUM_COLS, DIM, NUM_ROWS), jnp.float32),
    mesh=_mesh,
    scratch_types=[
        pltpu.VMEM((N_CHUNKS, CHUNK), jnp.int32),
        pltpu.VMEM((N_CHUNKS, CHUNK), jnp.int32),
        pltpu.VMEM((CHUNK, 128), jnp.float32),
        pltpu.VMEM((CHUNK, 128), jnp.float32),
        pltpu.VMEM((DIM, CHUNK), jnp.float32),
        pltpu.VMEM((DIM, CHUNK), jnp.float32),
        pltpu.SemaphoreType.DMA,
        pltpu.SemaphoreType.DMA,
        pltpu.SemaphoreType.DMA,
        pltpu.SemaphoreType.DMA,
    ],
    compiler_params=pltpu.CompilerParams(use_tc_tiling_on_sc=False,
                                         needs_layout_passes=False),
)
def _emb_lookup(idx4_hbm, sub_hbm, table_hbm, out_hbm, idx4_v, sub_v,
                buf_a, buf_b, ot_a, ot_b, gsem_a, gsem_b, ssem_a, ssem_b):
    wid = lax.axis_index("s") * NC + lax.axis_index("c")
    pltpu.sync_copy(idx4_hbm.at[wid], idx4_v)
    pltpu.sync_copy(sub_hbm.at[wid], sub_v)

    def fire_gather(i, buf, gsem):
        pltpu.async_copy(table_hbm.at[idx4_v.at[i]], buf, gsem)

    def drain_gather(i, buf, gsem):
        pltpu.make_async_copy(table_hbm.at[idx4_v.at[i]], buf, gsem).wait()

    def out_slice(i):
        c = wid * N_CHUNKS + i
        j = c >> 7
        b0 = (c & 127) * CHUNK
        return out_hbm.at[j, :, pl.ds(b0, CHUNK)]

    def fire_store(i, ot, ssem):
        pltpu.async_copy(ot, out_slice(i), ssem)

    def drain_store(ot, ssem):
        pltpu.make_async_copy(ot, out_hbm.at[0, :, pl.ds(0, CHUNK)],
                              ssem).wait()

    def extract(i, buf, ot):
        def per_group(g, carry):
            rows = g * LANES + lax.iota(jnp.int32, LANES)
            col0 = sub_v.at[i][pl.ds(g * LANES, LANES)]
            for d in range(DIM):
                vals = plsc.load_gather(buf, [rows, col0 + d])
                ot[d, pl.ds(g * LANES, LANES)] = vals
            return carry

        lax.fori_loop(0, CHUNK // LANES, per_group, 0)

    fire_gather(0, buf_a, gsem_a)

    def body(g2, carry):
        ca = 2 * g2
        cb = ca + 1
        drain_gather(ca, buf_a, gsem_a)

        @pl.when(g2 > 0)
        def _():
            drain_store(ot_b, ssem_b)

        fire_gather(cb, buf_b, gsem_b)
        extract(ca, buf_a, ot_a)
        fire_store(ca, ot_a, ssem_a)

        @pl.when(g2 < G2 - 1)
        def _():
            fire_gather(ca + 2, buf_a, gsem_a)

        drain_gather(cb, buf_b, gsem_b)
        drain_store(ot_a, ssem_a)
        extract(cb, buf_b, ot_b)
        fire_store(cb, ot_b, ssem_b)
        return carry

    lax.fori_loop(0, G2, body, 0)
    drain_store(ot_b, ssem_b)


def kernel(input, weight):
    idx_t = jnp.swapaxes(input, 0, 1).reshape(NW, N_CHUNKS, CHUNK)
    idx4 = idx_t >> 2
    sub = (idx_t & 3) * DIM
    w4 = weight.reshape(NUM_EMB * DIM // 128, 128)
    out = _emb_lookup(idx4, sub, w4)
    return jnp.transpose(out, (2, 0, 1))


# speedup vs baseline: 1.2148x; 1.2148x over previous
import functools

import jax
import jax.numpy as jnp
from jax import lax
from jax.experimental import pallas as pl
from jax.experimental.pallas import tpu as pltpu
from jax.experimental.pallas import tpu_sc as plsc

NUM_ROWS = 16384
NUM_COLS = 26
DIM = 32

B_TOTAL = NUM_ROWS * NUM_COLS
NUM_EMB = 1000000
NC = 2
NS = 16
NW = NC * NS
B_PER_W = B_TOTAL // NW
CHUNK = 128
N_CHUNKS = B_PER_W // CHUNK
NBUF = 4
N_GROUPS = N_CHUNKS // NBUF
G2 = N_GROUPS // 2

_mesh = plsc.VectorSubcoreMesh(core_axis_name="c", subcore_axis_name="s")


@functools.partial(
    pl.kernel,
    out_type=jax.ShapeDtypeStruct((B_TOTAL, DIM), jnp.float32),
    mesh=_mesh,
    scratch_types=[
        pltpu.VMEM((N_CHUNKS, CHUNK), jnp.int32),
        pltpu.VMEM((NBUF, CHUNK, DIM), jnp.float32),
        pltpu.VMEM((NBUF, CHUNK, DIM), jnp.float32),
        pltpu.SemaphoreType.DMA,
        pltpu.SemaphoreType.DMA,
        pltpu.SemaphoreType.DMA,
        pltpu.SemaphoreType.DMA,
    ],
    compiler_params=pltpu.CompilerParams(use_tc_tiling_on_sc=False),
)
def _emb_lookup(idx_hbm, table_hbm, out_hbm, idx_v, buf_a, buf_b,
                gsem_a, gsem_b, ssem_a, ssem_b):
    wid = lax.axis_index("s") * NC + lax.axis_index("c")
    base = wid * B_PER_W
    pltpu.sync_copy(idx_hbm.at[wid], idx_v)

    def fire_gathers(g, buf, gsem):
        for b in range(NBUF):
            i = g * NBUF + b
            pltpu.async_copy(table_hbm.at[idx_v.at[i]], buf.at[b], gsem)

    def drain_gathers(g, buf, gsem):
        for b in range(NBUF):
            i = g * NBUF + b
            pltpu.make_async_copy(table_hbm.at[idx_v.at[i]], buf.at[b],
                                  gsem).wait()

    def fire_stores(g, buf, ssem):
        for b in range(NBUF):
            i = g * NBUF + b
            pltpu.async_copy(buf.at[b],
                             out_hbm.at[pl.ds(base + i * CHUNK, CHUNK)], ssem)

    def drain_stores(buf, ssem):
        for b in range(NBUF):
            pltpu.make_async_copy(buf.at[b], out_hbm.at[pl.ds(base, CHUNK)],
                                  ssem).wait()

    fire_gathers(0, buf_a, gsem_a)

    def body(g2, carry):
        ga = 2 * g2
        gb = ga + 1
        drain_gathers(ga, buf_a, gsem_a)

        @pl.when(g2 > 0)
        def _():
            drain_stores(buf_b, ssem_b)

        fire_gathers(gb, buf_b, gsem_b)
        fire_stores(ga, buf_a, ssem_a)
        drain_gathers(gb, buf_b, gsem_b)
        drain_stores(buf_a, ssem_a)

        @pl.when(g2 < G2 - 1)
        def _():
            fire_gathers(ga + 2, buf_a, gsem_a)

        fire_stores(gb, buf_b, ssem_b)
        return carry

    lax.fori_loop(0, G2, body, 0)
    drain_stores(buf_b, ssem_b)


def kernel(input, weight):
    idx = jnp.swapaxes(input, 0, 1).reshape(NW, N_CHUNKS, CHUNK)
    out = _emb_lookup(idx, weight)
    out3 = out.reshape(NUM_COLS, NUM_ROWS, DIM)
    return jnp.swapaxes(out3, 0, 1)
